# baseline (speedup 1.0000x reference)
import functools

import jax
import jax.numpy as jnp
from jax import lax
from jax.experimental import pallas as pl
from jax.experimental.pallas import tpu as pltpu
from jax.experimental.pallas import tpu_sc as plsc

_STEPS = 128
_X_MIN = 0.0
_X_MAX = 1.0
_L = 16
_R = 256
_NBUF = 2


def _make_sc_kernel(n_rows: int):
    info = plsc.get_sparse_core_info()
    nc, ns = info.num_cores, info.num_subcores
    nw = nc * ns
    assert n_rows % (nw * _R) == 0
    rows_per_w = n_rows // nw
    chunks_per_w = rows_per_w // _R
    chunk_elems = _R * _STEPS

    mesh = plsc.VectorSubcoreMesh(core_axis_name="c", subcore_axis_name="s")

    @functools.partial(
        pl.kernel,
        mesh=mesh,
        out_type=jax.ShapeDtypeStruct((n_rows * _STEPS,), jnp.float32),
        compiler_params=pltpu.CompilerParams(needs_layout_passes=False),
        scratch_types=[
            pltpu.VMEM((rows_per_w,), jnp.float32),
            pltpu.VMEM((chunk_elems,), jnp.float32),
            pltpu.VMEM((chunk_elems,), jnp.float32),
            pltpu.VMEM((_R,), jnp.int32),
            pltpu.VMEM((_R,), jnp.int32),
            pltpu.SemaphoreType.DMA,
            pltpu.SemaphoreType.DMA,
        ],
    )
    def k(x_hbm, out_hbm, x_v, buf0, buf1, pos0, pos1, sem0, sem1):
        wid = lax.axis_index("s") * nc + lax.axis_index("c")
        row_base = wid * rows_per_w

        pltpu.sync_copy(x_hbm.at[pl.ds(row_base, rows_per_w)], x_v)

        lane = lax.iota(jnp.int32, _L)
        lane_off = lane * _STEPS
        ones = jnp.full((_L,), 1.0, jnp.float32)
        zeros = jnp.zeros((_L,), jnp.float32)

        def _zero(i, _):
            buf0[pl.ds(i * _L, _L)] = zeros
            buf1[pl.ds(i * _L, _L)] = zeros
            return 0
        lax.fori_loop(0, chunk_elems // _L, _zero, 0)

        bufs = (buf0, buf1)
        poss = (pos0, pos1)
        sems = (sem0, sem1)

        def _chunk(c, buf, posv, sem):
            rel = c * _R

            @pl.when(c >= _NBUF)
            def _():
                pltpu.make_async_copy(
                    buf, out_hbm.at[pl.ds((row_base + rel) * _STEPS,
                                          chunk_elems)], sem).wait()
                for g in range(_R // _L):
                    pv = posv[pl.ds(g * _L, _L)]
                    plsc.store_scatter(buf, [pv], zeros)

            for g in range(_R // _L):
                xv = x_v[pl.ds(rel + g * _L, _L)]
                xs = (xv - _X_MIN) * ((_STEPS - 1) / (_X_MAX - _X_MIN))
                idx = jnp.clip(xs, 0.0, float(_STEPS - 1)).astype(jnp.int32)
                pos = idx + (g * _L) * _STEPS + lane_off
                posv[pl.ds(g * _L, _L)] = pos
                plsc.store_scatter(buf, [pos], ones)

            pltpu.make_async_copy(
                buf, out_hbm.at[pl.ds((row_base + rel) * _STEPS,
                                      chunk_elems)], sem).start()

        def _step(t, _):
            for b in range(_NBUF):
                _chunk(t * _NBUF + b, bufs[b], poss[b], sems[b])
            return 0
        lax.fori_loop(0, chunks_per_w // _NBUF, _step, 0)

        for b in range(_NBUF):
            pltpu.make_async_copy(
                bufs[b], out_hbm.at[pl.ds(row_base * _STEPS, chunk_elems)],
                sems[b]).wait()

    return k


def kernel(x, eye):
    del eye
    n, c = x.shape
    n_rows = n * c
    out_flat = _make_sc_kernel(n_rows)(x.reshape(n_rows))
    return out_flat.reshape(n, c, _STEPS)
